# Initial kernel scaffold; baseline (speedup 1.0000x reference)
import functools

import jax
import jax.numpy as jnp
from jax import lax
from jax.experimental import pallas as pl
from jax.experimental.pallas import tpu as pltpu
from jax.experimental.pallas import tpu_sc as plsc

N, E, D, ED, G = 10000, 320000, 128, 16, 64
NLAYER, LS = 3, 2
NP = 10240
NC, NS = 2, 16
NW = NC * NS
EPT = E // NW
C = 80
NCHUNK = EPT // C
RPT = NP // NS
LEAK = 0.01
BNC = 1.0 / (1.0 + 1e-5) ** 0.5


def _sc_agg_body(lidx, x_hbm, e_hbm, src_hbm, dst_hbm, out_hbm,
                 srcv, dstv, xrows, erows, agg, sem):
    c = lax.axis_index("c")
    s = lax.axis_index("s")
    wid = c * NS + s
    rbase = s * RPT
    pltpu.sync_copy(x_hbm.at[pl.ds(rbase, RPT)], agg.at[pl.ds(rbase, RPT)])
    plsc.subcore_barrier()
    tbase = wid * EPT

    def chunk(i, carry):
        base = pl.multiple_of(tbase + i * C, 8)
        pltpu.sync_copy(src_hbm.at[pl.ds(base, C)], srcv)
        pltpu.sync_copy(dst_hbm.at[pl.ds(base, C)], dstv)
        cp = pltpu.async_copy(x_hbm.at[srcv], xrows, sem)
        pltpu.sync_copy(e_hbm.at[pl.ds(lidx * E + base, C)], erows)
        cp.wait()

        def row(r, cc):
            for k in range(D // 16):
                sl = pl.ds(k * 16, 16)
                xrows[r, sl] = jnp.maximum(xrows[r, sl] + erows[r, sl], 0.0)
            return cc

        lax.fori_loop(0, C, row, 0)
        pltpu.sync_copy(xrows, agg.at[dstv], add=True)
        return carry

    lax.fori_loop(0, NCHUNK, chunk, 0)
    plsc.subcore_barrier()
    pltpu.sync_copy(agg.at[pl.ds(rbase, RPT)], out_hbm.at[c, pl.ds(rbase, RPT)])


def _make_sc_agg(lidx):
    mesh = plsc.VectorSubcoreMesh(core_axis_name="c", subcore_axis_name="s")
    return pl.kernel(
        functools.partial(_sc_agg_body, lidx),
        out_type=jax.ShapeDtypeStruct((NC, NP, D), jnp.float32),
        mesh=mesh,
        scratch_types=[
            pltpu.VMEM((C,), jnp.int32),
            pltpu.VMEM((C,), jnp.int32),
            pltpu.VMEM((C, D), jnp.float32),
            pltpu.VMEM((C, D), jnp.float32),
            pltpu.VMEM_SHARED((NP, D), jnp.float32),
            pltpu.SemaphoreType.DMA,
        ],
    )


def _dot_t(a, b):
    return lax.dot_general(a, b, (((1,), (1,)), ((), ())),
                           preferred_element_type=jnp.float32)


def _leaky(x, s=LEAK):
    return jnp.where(x >= 0, x, s * x)


def _elu(x):
    return jnp.where(x > 0, x, jnp.exp(x) - 1.0)


BE = 4000


def _edge_body(attr_ref, w_ref, b_ref, out_ref):
    out_ref[...] = _dot_t(attr_ref[...], w_ref[0]) + b_ref[0]


def _edge_mm(attr, w3, b3):
    nb = E // BE
    return pl.pallas_call(
        _edge_body,
        grid=(3, nb),
        in_specs=[
            pl.BlockSpec((BE, ED), lambda l, i: (i, 0)),
            pl.BlockSpec((1, D, ED), lambda l, i: (l, 0, 0)),
            pl.BlockSpec((1, 1, D), lambda l, i: (l, 0, 0)),
        ],
        out_specs=pl.BlockSpec((BE, D), lambda l, i: (l * nb + i, 0)),
        out_shape=jax.ShapeDtypeStruct((3 * E, D), jnp.float32),
    )(attr, w3, b3)


BN = 1024


def _node_body(x_ref, pa_ref, w1_ref, b1_ref, w2_ref, b2_ref,
               wih_ref, bih_ref, whh_ref, bhh_ref, out_ref):
    x = x_ref[...]
    z = pa_ref[0] + pa_ref[1] - x
    h = (_dot_t(z, w1_ref[...]) + b1_ref[...]) * BNC
    h = _leaky(h)
    h = _dot_t(h, w2_ref[...]) + b2_ref[...]
    h = _elu(h)
    gi = _dot_t(h, wih_ref[...]) + bih_ref[...]
    gh = _dot_t(x, whh_ref[...]) + bhh_ref[...]
    r = jax.nn.sigmoid(gi[:, :D] + gh[:, :D])
    zg = jax.nn.sigmoid(gi[:, D:2 * D] + gh[:, D:2 * D])
    n = jnp.tanh(gi[:, 2 * D:] + r * gh[:, 2 * D:])
    out_ref[...] = _leaky((1.0 - zg) * n + zg * x)


def _node_update(x, pa, w1, b1, w2, b2, wih, bih, whh, bhh):
    nb = NP // BN
    full = lambda shape: pl.BlockSpec(shape, lambda i: tuple(0 for _ in shape))
    return pl.pallas_call(
        _node_body,
        grid=(nb,),
        in_specs=[
            pl.BlockSpec((BN, D), lambda i: (i, 0)),
            pl.BlockSpec((NC, BN, D), lambda i: (0, i, 0)),
            full((D, D)), full((1, D)),
            full((D, D)), full((1, D)),
            full((3 * D, D)), full((1, 3 * D)),
            full((3 * D, D)), full((1, 3 * D)),
        ],
        out_specs=pl.BlockSpec((BN, D), lambda i: (i, 0)),
        out_shape=jax.ShapeDtypeStruct((NP, D), jnp.float32),
    )(x, pa, w1, b1, w2, b2, wih, bih, whh, bhh)


def _gat_body(x_ref, b_ref, wl_ref, bl_ref, wr_ref, br_ref, att_ref, bias_ref,
              wih_ref, bih_ref, whh_ref, bhh_ref, linw_ref, linb_ref,
              res_ref, att_out_ref):
    x = x_ref[...]
    bidx = b_ref[...]
    gids = lax.broadcasted_iota(jnp.int32, (G, NP), 0)
    M = (gids == bidx).astype(jnp.float32)
    dg = lambda a, b, da, db: lax.dot_general(
        a, b, (((da,), (db,)), ((), ())), preferred_element_type=jnp.float32)
    out = _leaky(dg(M, x, 1, 0))
    xl = _dot_t(x, wl_ref[...]) + bl_ref[...]
    attsum = jnp.zeros((1, NP), jnp.float32)
    for _ in range(LS):
        xr = _dot_t(out, wr_ref[...]) + br_ref[...]
        xrn = dg(M, xr, 0, 0)
        eatt = _leaky(xl + xrn, 0.2)
        logits = dg(att_ref[...], eatt, 1, 1)
        masked = jnp.where(M > 0, logits, -1e30)
        mx = jnp.max(masked, axis=1, keepdims=True)
        mxn = dg(mx, M, 0, 0)
        ex = jnp.exp(logits - mxn)
        ssum = dg(M, ex, 1, 1)
        rcp = jnp.where(ssum > 0, 1.0 / ssum, 0.0)
        rcpn = dg(rcp, M, 0, 0)
        alpha = ex * rcpn
        h = dg(M * alpha, xl, 1, 0) + bias_ref[...]
        h = _elu(h)
        gi = _dot_t(h, wih_ref[...]) + bih_ref[...]
        gh = _dot_t(out, whh_ref[...]) + bhh_ref[...]
        r = jax.nn.sigmoid(gi[:, :D] + gh[:, :D])
        zg = jax.nn.sigmoid(gi[:, D:2 * D] + gh[:, D:2 * D])
        n = jnp.tanh(gi[:, 2 * D:] + r * gh[:, 2 * D:])
        out = _leaky((1.0 - zg) * n + zg * out)
        attsum = attsum + alpha
    res_ref[...] = dg(linw_ref[...], out, 1, 1) + linb_ref[...]
    att_out_ref[...] = attsum * (1.0 / LS)


def _gat_head(x, batchp, wl, bl, wr, br, att, bias, wih, bih, whh, bhh,
              linw, linb):
    return pl.pallas_call(
        _gat_body,
        out_shape=[
            jax.ShapeDtypeStruct((1, G), jnp.float32),
            jax.ShapeDtypeStruct((1, NP), jnp.float32),
        ],
    )(x, batchp, wl, bl, wr, br, att, bias, wih, bih, whh, bhh, linw, linb)


def kernel(x_g, edge_attr_g, params, edge_index_g, batch):
    p = params
    xp = jnp.zeros((NP, D), jnp.float32).at[:N].set(x_g)
    src = edge_index_g[0]
    dst = edge_index_g[1]
    batchp = jnp.full((1, NP), G, jnp.int32).at[0, :N].set(batch)

    w3 = jnp.stack([p[f"l{l}_We"] for l in range(NLAYER)])
    b3 = jnp.stack([p[f"l{l}_be"] for l in range(NLAYER)])[:, None]
    e3 = _edge_mm(edge_attr_g, w3, b3)

    x = xp
    for l in range(NLAYER):
        pa = _make_sc_agg(l)(x, e3, src, dst)
        x = _node_update(
            x, pa,
            p[f"l{l}_W1"], p[f"l{l}_b1"][None],
            p[f"l{l}_W2"], p[f"l{l}_b2"][None],
            p[f"l{l}_Wih"], p[f"l{l}_bih"][None],
            p[f"l{l}_Whh"], p[f"l{l}_bhh"][None],
        )

    res, attsum = _gat_head(
        x, batchp,
        p["mol_Wl"], p["mol_bl"][None], p["mol_Wr"], p["mol_br"][None],
        p["mol_att"][None], p["mol_bias"][None],
        p["mol_Wih"], p["mol_bih"][None], p["mol_Whh"], p["mol_bhh"][None],
        p["lin_W"], p["lin_b"][None],
    )
    return res.reshape(G, 1), attsum[0, :N]


# speedup vs baseline: 2.9311x; 2.9311x over previous
import functools

import jax
import jax.numpy as jnp
from jax import lax
from jax.experimental import pallas as pl
from jax.experimental.pallas import tpu as pltpu
from jax.experimental.pallas import tpu_sc as plsc

N, E, D, ED, G = 10000, 320000, 128, 16, 64
NLAYER, LS = 3, 2
NP = 10240
NC, NS = 2, 16
NW = NC * NS
EPT = E // NW
C = 80
NCHUNK = EPT // C
RPT = NP // NS
LEAK = 0.01
BNC = 1.0 / (1.0 + 1e-5) ** 0.5


def _sc_agg_body(lidx, x_hbm, e_hbm, src_hbm, dst_hbm, out_hbm,
                 srcv, dstv, xrows, erows, agg, sem):
    c = lax.axis_index("c")
    s = lax.axis_index("s")
    wid = c * NS + s
    rbase = s * RPT
    pltpu.sync_copy(x_hbm.at[pl.ds(rbase, RPT)], agg.at[pl.ds(rbase, RPT)])
    plsc.subcore_barrier()
    tbase = wid * EPT

    def chunk(i, carry):
        base = pl.multiple_of(tbase + i * C, 8)
        pltpu.sync_copy(src_hbm.at[pl.ds(base, C)], srcv)
        pltpu.sync_copy(dst_hbm.at[pl.ds(base, C)], dstv)
        cp = pltpu.async_copy(x_hbm.at[srcv], xrows, sem)
        pltpu.sync_copy(e_hbm.at[pl.ds(lidx * E + base, C)], erows)
        cp.wait()

        def row(r, cc):
            for k in range(D // 16):
                sl = pl.ds(k * 16, 16)
                xrows[r, sl] = jnp.maximum(xrows[r, sl] + erows[r, sl], 0.0)
            return cc

        lax.fori_loop(0, C, row, 0)
        pltpu.sync_copy(xrows, agg.at[dstv], add=True)
        return carry

    lax.fori_loop(0, NCHUNK, chunk, 0)
    plsc.subcore_barrier()
    pltpu.sync_copy(agg.at[pl.ds(rbase, RPT)], out_hbm.at[c, pl.ds(rbase, RPT)])


def _make_sc_agg(lidx):
    mesh = plsc.VectorSubcoreMesh(core_axis_name="c", subcore_axis_name="s")
    return pl.kernel(
        functools.partial(_sc_agg_body, lidx),
        out_type=jax.ShapeDtypeStruct((NC, NP, D), jnp.float32),
        mesh=mesh,
        scratch_types=[
            pltpu.VMEM((C,), jnp.int32),
            pltpu.VMEM((C,), jnp.int32),
            pltpu.VMEM((C, D), jnp.float32),
            pltpu.VMEM((C, D), jnp.float32),
            pltpu.VMEM_SHARED((NP, D), jnp.float32),
            pltpu.SemaphoreType.DMA,
        ],
    )


def _dot_t(a, b):
    return lax.dot_general(a, b, (((1,), (1,)), ((), ())),
                           preferred_element_type=jnp.float32)


def _leaky(x, s=LEAK):
    return jnp.where(x >= 0, x, s * x)


def _elu(x):
    return jnp.where(x > 0, x, jnp.exp(x) - 1.0)


BE = 4000


def _edge_body(attr_ref, w_ref, b_ref, out_ref):
    out_ref[...] = _dot_t(attr_ref[...], w_ref[0]) + b_ref[0]


def _edge_mm(attr, w3, b3):
    nb = E // BE
    return pl.pallas_call(
        _edge_body,
        grid=(3, nb),
        in_specs=[
            pl.BlockSpec((BE, ED), lambda l, i: (i, 0)),
            pl.BlockSpec((1, D, ED), lambda l, i: (l, 0, 0)),
            pl.BlockSpec((1, 1, D), lambda l, i: (l, 0, 0)),
        ],
        out_specs=pl.BlockSpec((BE, D), lambda l, i: (l * nb + i, 0)),
        out_shape=jax.ShapeDtypeStruct((3 * E, D), jnp.float32),
    )(attr, w3, b3)


BN = 1024


def _node_body(x_ref, pa_ref, w1_ref, b1_ref, w2_ref, b2_ref,
               wih_ref, bih_ref, whh_ref, bhh_ref, out_ref):
    x = x_ref[...]
    z = pa_ref[0] + pa_ref[1] - x
    h = (_dot_t(z, w1_ref[...]) + b1_ref[...]) / jnp.sqrt(jnp.float32(1.0 + 1e-5))
    h = _leaky(h)
    h = _dot_t(h, w2_ref[...]) + b2_ref[...]
    h = _elu(h)
    gi = _dot_t(h, wih_ref[...]) + bih_ref[...]
    gh = _dot_t(x, whh_ref[...]) + bhh_ref[...]
    r = jax.nn.sigmoid(gi[:, :D] + gh[:, :D])
    zg = jax.nn.sigmoid(gi[:, D:2 * D] + gh[:, D:2 * D])
    n = jnp.tanh(gi[:, 2 * D:] + r * gh[:, 2 * D:])
    out_ref[...] = _leaky((1.0 - zg) * n + zg * x)


def _node_update(x, pa, w1, b1, w2, b2, wih, bih, whh, bhh):
    nb = NP // BN
    full = lambda shape: pl.BlockSpec(shape, lambda i: tuple(0 for _ in shape))
    return pl.pallas_call(
        _node_body,
        grid=(nb,),
        in_specs=[
            pl.BlockSpec((BN, D), lambda i: (i, 0)),
            pl.BlockSpec((NC, BN, D), lambda i: (0, i, 0)),
            full((D, D)), full((1, D)),
            full((D, D)), full((1, D)),
            full((3 * D, D)), full((1, 3 * D)),
            full((3 * D, D)), full((1, 3 * D)),
        ],
        out_specs=pl.BlockSpec((BN, D), lambda i: (i, 0)),
        out_shape=jax.ShapeDtypeStruct((NP, D), jnp.float32),
    )(x, pa, w1, b1, w2, b2, wih, bih, whh, bhh)


def _pool_xl_body(x_ref, b_ref, wl_ref, bl_ref, xl_ref, pool_ref):
    i = pl.program_id(0)
    x = x_ref[...]
    xl_ref[...] = _dot_t(x, wl_ref[...]) + bl_ref[...]
    gids = lax.broadcasted_iota(jnp.int32, (G, BN), 0)
    M = (gids == b_ref[...]).astype(jnp.float32)
    part = lax.dot_general(M, x, (((1,), (0,)), ((), ())),
                           precision=lax.Precision.HIGHEST,
                           preferred_element_type=jnp.float32)

    @pl.when(i == 0)
    def _():
        pool_ref[...] = part

    @pl.when(i != 0)
    def _():
        pool_ref[...] = pool_ref[...] + part


def _pool_xl(x, batchp, wl, bl):
    nb = NP // BN
    full = lambda shape: pl.BlockSpec(shape, lambda i: tuple(0 for _ in shape))
    return pl.pallas_call(
        _pool_xl_body,
        grid=(nb,),
        in_specs=[
            pl.BlockSpec((BN, D), lambda i: (i, 0)),
            pl.BlockSpec((1, BN), lambda i: (0, i)),
            full((D, D)), full((1, D)),
        ],
        out_specs=[
            pl.BlockSpec((BN, D), lambda i: (i, 0)),
            pl.BlockSpec((G, D), lambda i: (0, 0)),
        ],
        out_shape=[
            jax.ShapeDtypeStruct((NP, D), jnp.float32),
            jax.ShapeDtypeStruct((G, D), jnp.float32),
        ],
    )(x, batchp, wl, bl)


def _gat_body(xl_ref, pool_ref, b_ref, wr_ref, br_ref, att_ref, bias_ref,
              wih_ref, bih_ref, whh_ref, bhh_ref, linw_ref, linb_ref,
              res_ref, att_out_ref):
    bidx = b_ref[...]
    gids = lax.broadcasted_iota(jnp.int32, (G, NP), 0)
    M = (gids == bidx).astype(jnp.float32)
    dg = lambda a, b, da, db: lax.dot_general(
        a, b, (((da,), (db,)), ((), ())), precision=lax.Precision.HIGHEST,
        preferred_element_type=jnp.float32)
    out = _leaky(pool_ref[...])
    xl = xl_ref[...]
    attsum = jnp.zeros((1, NP), jnp.float32)
    for _ in range(LS):
        xr = _dot_t(out, wr_ref[...]) + br_ref[...]
        xrn = dg(M, xr, 0, 0)
        eatt = _leaky(xl + xrn, 0.2)
        logits = _dot_t(att_ref[...], eatt)
        masked = jnp.where(M > 0, logits, -1e30)
        mx = jnp.max(masked, axis=1, keepdims=True)
        mxn = dg(mx, M, 0, 0)
        ex = jnp.exp(logits - mxn)
        ssum = dg(M, ex, 1, 1)
        rcp = jnp.where(ssum > 0, 1.0 / ssum, 0.0)
        rcpn = dg(rcp, M, 0, 0)
        alpha = ex * rcpn
        h = dg(M * alpha, xl, 1, 0) + bias_ref[...]
        h = _elu(h)
        gi = _dot_t(h, wih_ref[...]) + bih_ref[...]
        gh = _dot_t(out, whh_ref[...]) + bhh_ref[...]
        r = jax.nn.sigmoid(gi[:, :D] + gh[:, :D])
        zg = jax.nn.sigmoid(gi[:, D:2 * D] + gh[:, D:2 * D])
        n = jnp.tanh(gi[:, 2 * D:] + r * gh[:, 2 * D:])
        out = _leaky((1.0 - zg) * n + zg * out)
        attsum = attsum + alpha
    res_ref[...] = _dot_t(linw_ref[...], out) + linb_ref[...]
    att_out_ref[...] = attsum * (1.0 / LS)


def _gat_head(x, batchp, wl, bl, wr, br, att, bias, wih, bih, whh, bhh,
              linw, linb):
    xl, pool = _pool_xl(x, batchp, wl, bl)
    return pl.pallas_call(
        _gat_body,
        out_shape=[
            jax.ShapeDtypeStruct((1, G), jnp.float32),
            jax.ShapeDtypeStruct((1, NP), jnp.float32),
        ],
    )(xl, pool, batchp, wr, br, att, bias, wih, bih, whh, bhh, linw, linb)


def kernel(x_g, edge_attr_g, params, edge_index_g, batch):
    p = params
    xp = jnp.zeros((NP, D), jnp.float32).at[:N].set(x_g)
    src = edge_index_g[0]
    dst = edge_index_g[1]
    batchp = jnp.full((1, NP), G, jnp.int32).at[0, :N].set(batch)

    w3 = jnp.stack([p[f"l{l}_We"] for l in range(NLAYER)])
    b3 = jnp.stack([p[f"l{l}_be"] for l in range(NLAYER)])[:, None]
    e3 = _edge_mm(edge_attr_g, w3, b3)

    x = xp
    for l in range(NLAYER):
        pa = _make_sc_agg(l)(x, e3, src, dst)
        x = _node_update(
            x, pa,
            p[f"l{l}_W1"], p[f"l{l}_b1"][None],
            p[f"l{l}_W2"], p[f"l{l}_b2"][None],
            p[f"l{l}_Wih"], p[f"l{l}_bih"][None],
            p[f"l{l}_Whh"], p[f"l{l}_bhh"][None],
        )

    res, attsum = _gat_head(
        x, batchp,
        p["mol_Wl"], p["mol_bl"][None], p["mol_Wr"], p["mol_br"][None],
        p["mol_att"][None], p["mol_bias"][None],
        p["mol_Wih"], p["mol_bih"][None], p["mol_Whh"], p["mol_bhh"][None],
        p["lin_W"], p["lin_b"][None],
    )
    return res.reshape(G, 1), attsum[0, :N]
